# baseline (speedup 1.0000x reference)
import functools

import jax
import jax.numpy as jnp
from jax import lax
from jax.experimental import pallas as pl
from jax.experimental.pallas import tpu as pltpu
from jax.experimental.pallas import tpu_sc as plsc

_NC = 2
_NS = 16
_NW = _NC * _NS
_CHUNK = 128


def _sc_gather(flat, idx, ncols, B):
    b_per_w = B // _NW
    n_chunks = b_per_w // _CHUNK

    mesh = plsc.VectorSubcoreMesh(core_axis_name="c", subcore_axis_name="s")

    @functools.partial(
        pl.kernel,
        mesh=mesh,
        out_type=jax.ShapeDtypeStruct((3, 1, B), jnp.float32),
        scratch_types=[
            pltpu.VMEM((b_per_w,), jnp.int32),
            pltpu.VMEM((b_per_w,), jnp.int32),
            pltpu.VMEM((b_per_w,), jnp.int32),
            pltpu.VMEM((b_per_w,), jnp.int32),
            pltpu.VMEM((b_per_w,), jnp.float32),
            pltpu.VMEM((b_per_w,), jnp.float32),
            pltpu.VMEM((b_per_w,), jnp.float32),
            pltpu.SemaphoreType.DMA,
        ],
    )
    def gather_kernel(flat_hbm, idx_hbm, out,
                      idx_v, s0, s1, s2, bx, by, bz, sem):
        wid = lax.axis_index("s") * _NC + lax.axis_index("c")
        base = wid * b_per_w
        pltpu.sync_copy(idx_hbm.at[pl.ds(base, b_per_w)], idx_v)
        nc = jnp.int32(ncols)
        for j in range(b_per_w // 16):
            sl = pl.ds(j * 16, 16)
            v = idx_v[sl] * nc
            s0[sl] = v
            s1[sl] = v + 1
            s2[sl] = v + 2
        copies = []
        for sidx, buf in ((s0, bx), (s1, by), (s2, bz)):
            for j in range(n_chunks):
                sl = pl.ds(j * _CHUNK, _CHUNK)
                copies.append(pltpu.async_copy(flat_hbm.at[sidx.at[sl]],
                                               buf.at[sl], sem))
        for cp in copies:
            cp.wait()
        pltpu.sync_copy(bx, out.at[0, 0, pl.ds(base, b_per_w)])
        pltpu.sync_copy(by, out.at[1, 0, pl.ds(base, b_per_w)])
        pltpu.sync_copy(bz, out.at[2, 0, pl.ds(base, b_per_w)])

    return gather_kernel(flat, idx)


def _jacobi_rotate(Am, Vm, p, q):
    r = 3 - p - q
    app, aqq, apq = Am[p][p], Am[q][q], Am[p][q]
    apr, aqr = Am[p][r], Am[q][r]

    apq_zero = apq == 0.0
    apq_safe = jnp.where(apq_zero, 1.0, apq)
    tau = (aqq - app) * 0.5 / apq_safe
    sgn = jnp.where(tau >= 0.0, 1.0, -1.0)
    t = sgn / (jnp.abs(tau) + jnp.sqrt(1.0 + tau * tau))
    t = jnp.where(apq_zero, 0.0, t)
    c = lax.rsqrt(1.0 + t * t)
    s = t * c

    Am[p][p] = app - t * apq
    Am[q][q] = aqq + t * apq
    zero = apq * 0.0
    Am[p][q] = zero
    Am[q][p] = zero
    npr = c * apr - s * aqr
    nqr = c * aqr + s * apr
    Am[p][r] = npr
    Am[r][p] = npr
    Am[q][r] = nqr
    Am[r][q] = nqr
    for i in range(3):
        vip, viq = Vm[i][p], Vm[i][q]
        Vm[i][p] = c * vip - s * viq
        Vm[i][q] = s * vip + c * viq


def _feats_body(g_ref, o_ref):
    X = g_ref[0]
    Y = g_ref[1]
    Z = g_ref[2]
    S = X.shape[1]
    inv = jnp.float32(1.0 / S)

    cx = jnp.sum(X, axis=1, keepdims=True) * inv
    cy = jnp.sum(Y, axis=1, keepdims=True) * inv
    cz = jnp.sum(Z, axis=1, keepdims=True) * inv
    Xc = X - cx
    Yc = Y - cy
    Zc = Z - cz

    axx = jnp.sum(Xc * Xc, axis=1, keepdims=True)
    ayy = jnp.sum(Yc * Yc, axis=1, keepdims=True)
    azz = jnp.sum(Zc * Zc, axis=1, keepdims=True)
    axy = jnp.sum(Xc * Yc, axis=1, keepdims=True)
    axz = jnp.sum(Xc * Zc, axis=1, keepdims=True)
    ayz = jnp.sum(Yc * Zc, axis=1, keepdims=True)

    Am = [[axx, axy, axz], [axy, ayy, ayz], [axz, ayz, azz]]
    one = jnp.ones_like(axx)
    zer = jnp.zeros_like(axx)
    Vm = [[one, zer, zer], [zer, one, zer], [zer, zer, one]]
    for _ in range(6):
        _jacobi_rotate(Am, Vm, 0, 1)
        _jacobi_rotate(Am, Vm, 0, 2)
        _jacobi_rotate(Am, Vm, 1, 2)

    wa, wb, wc = Am[0][0], Am[1][1], Am[2][2]
    w2 = jnp.maximum(jnp.maximum(wa, wb), wc)
    w0 = jnp.minimum(jnp.minimum(wa, wb), wc)
    w1 = wa + wb + wc - w2 - w0

    a_max = jnp.logical_and(wa >= wb, wa >= wc)
    b_max = jnp.logical_and(jnp.logical_not(a_max), wb >= wc)
    v2x = jnp.where(a_max, Vm[0][0], jnp.where(b_max, Vm[0][1], Vm[0][2]))
    v2y = jnp.where(a_max, Vm[1][0], jnp.where(b_max, Vm[1][1], Vm[1][2]))
    v2z = jnp.where(a_max, Vm[2][0], jnp.where(b_max, Vm[2][1], Vm[2][2]))

    dirwt = 1.0 - w1 / w2
    iw2 = 1.0 / w2

    x0 = Xc * v2x + Yc * v2y + Zc * v2z
    r2 = Xc * Xc + Yc * Yc + Zc * Zc - x0 * x0
    np0 = jnp.sqrt(jnp.maximum(r2, 0.0))
    sc = jnp.sum(x0 * np0, axis=1, keepdims=True)
    flip = jnp.where(sc < 0.0, -dirwt, dirwt)
    v0x = flip * v2x
    v0y = flip * v2y
    v0z = flip * v2z

    size = jnp.full_like(axx, float(S))
    o_ref[...] = jnp.concatenate(
        [cx, cy, cz,
         axx * iw2, axy * iw2, axz * iw2,
         axy * iw2, ayy * iw2, ayz * iw2,
         axz * iw2, ayz * iw2, azz * iw2,
         v0x, v0y, v0z, size],
        axis=1,
    )


def _tc_feats(g):
    n = g.shape[1]
    return pl.pallas_call(
        _feats_body,
        out_shape=jax.ShapeDtypeStruct((n, 16), jnp.float32),
    )(g)


def kernel(data, clusts):
    n_clusts, S = clusts.shape
    n_vox, ncols = data.shape
    flat = data.reshape(-1).astype(jnp.float32)
    idx = clusts.reshape(-1).astype(jnp.int32)
    g = _sc_gather(flat, idx, ncols, n_clusts * S)
    return _tc_feats(g.reshape(3, n_clusts, S))


# speedup vs baseline: 1.8708x; 1.8708x over previous
import functools

import jax
import jax.numpy as jnp
from jax import lax
from jax.experimental import pallas as pl
from jax.experimental.pallas import tpu as pltpu
from jax.experimental.pallas import tpu_sc as plsc

_NC = 2
_NS = 16
_NW = _NC * _NS
_CHUNK = 128


def _sc_gather(xs, ys, zs, idx):
    B = idx.shape[0]
    N = xs.shape[0]
    b_per_w = B // _NW
    n_chunks = b_per_w // _CHUNK
    n_stage = N // _NS

    mesh = plsc.VectorSubcoreMesh(core_axis_name="c", subcore_axis_name="s")

    @functools.partial(
        pl.kernel,
        mesh=mesh,
        out_type=[jax.ShapeDtypeStruct((B,), jnp.float32)] * 3,
        scratch_types=[
            pltpu.VMEM_SHARED((N,), jnp.float32),
            pltpu.VMEM_SHARED((N,), jnp.float32),
            pltpu.VMEM_SHARED((N,), jnp.float32),
            pltpu.VMEM((b_per_w,), jnp.int32),
            pltpu.VMEM((b_per_w,), jnp.float32),
            pltpu.VMEM((b_per_w,), jnp.float32),
            pltpu.VMEM((b_per_w,), jnp.float32),
            pltpu.SemaphoreType.DMA,
            pltpu.SemaphoreType.DMA,
        ],
    )
    def gather_kernel(xs_hbm, ys_hbm, zs_hbm, idx_hbm, ox, oy, oz,
                      spx, spy, spz, idx_v, bx, by, bz, sem, osem):
        cid = lax.axis_index("c")
        sid = lax.axis_index("s")
        wid = sid * _NC + cid
        base = wid * b_per_w
        st = sid * n_stage
        stage = [
            pltpu.async_copy(xs_hbm.at[pl.ds(st, n_stage)],
                             spx.at[pl.ds(st, n_stage)], sem),
            pltpu.async_copy(ys_hbm.at[pl.ds(st, n_stage)],
                             spy.at[pl.ds(st, n_stage)], sem),
            pltpu.async_copy(zs_hbm.at[pl.ds(st, n_stage)],
                             spz.at[pl.ds(st, n_stage)], sem),
        ]
        pltpu.sync_copy(idx_hbm.at[pl.ds(base, b_per_w)], idx_v)
        for cp in stage:
            cp.wait()
        plsc.subcore_barrier()
        copies = []
        for tab, buf in ((spx, bx), (spy, by), (spz, bz)):
            for j in range(n_chunks):
                sl = pl.ds(j * _CHUNK, _CHUNK)
                copies.append(pltpu.async_copy(tab.at[idx_v.at[sl]],
                                               buf.at[sl], sem))
        for cp in copies:
            cp.wait()
        outs = [
            pltpu.async_copy(bx, ox.at[pl.ds(base, b_per_w)], osem),
            pltpu.async_copy(by, oy.at[pl.ds(base, b_per_w)], osem),
            pltpu.async_copy(bz, oz.at[pl.ds(base, b_per_w)], osem),
        ]
        for cp in outs:
            cp.wait()

    return gather_kernel(xs, ys, zs, idx)


def _jacobi_rotate(Am, Vm, p, q):
    r = 3 - p - q
    app, aqq, apq = Am[p][p], Am[q][q], Am[p][q]
    apr, aqr = Am[p][r], Am[q][r]

    apq_zero = apq == 0.0
    apq_safe = jnp.where(apq_zero, 1.0, apq)
    tau = (aqq - app) * 0.5 / apq_safe
    sgn = jnp.where(tau >= 0.0, 1.0, -1.0)
    t = sgn / (jnp.abs(tau) + jnp.sqrt(1.0 + tau * tau))
    t = jnp.where(apq_zero, 0.0, t)
    c = lax.rsqrt(1.0 + t * t)
    s = t * c

    Am[p][p] = app - t * apq
    Am[q][q] = aqq + t * apq
    zero = apq * 0.0
    Am[p][q] = zero
    Am[q][p] = zero
    npr = c * apr - s * aqr
    nqr = c * aqr + s * apr
    Am[p][r] = npr
    Am[r][p] = npr
    Am[q][r] = nqr
    Am[r][q] = nqr
    for i in range(3):
        vip, viq = Vm[i][p], Vm[i][q]
        Vm[i][p] = c * vip - s * viq
        Vm[i][q] = s * vip + c * viq


def _feats_body(x_ref, y_ref, z_ref, o_ref):
    X = x_ref[...]
    Y = y_ref[...]
    Z = z_ref[...]
    S = X.shape[1]
    inv = jnp.float32(1.0 / S)

    cx = jnp.sum(X, axis=1, keepdims=True) * inv
    cy = jnp.sum(Y, axis=1, keepdims=True) * inv
    cz = jnp.sum(Z, axis=1, keepdims=True) * inv
    Xc = X - cx
    Yc = Y - cy
    Zc = Z - cz

    axx = jnp.sum(Xc * Xc, axis=1, keepdims=True)
    ayy = jnp.sum(Yc * Yc, axis=1, keepdims=True)
    azz = jnp.sum(Zc * Zc, axis=1, keepdims=True)
    axy = jnp.sum(Xc * Yc, axis=1, keepdims=True)
    axz = jnp.sum(Xc * Zc, axis=1, keepdims=True)
    ayz = jnp.sum(Yc * Zc, axis=1, keepdims=True)

    Am = [[axx, axy, axz], [axy, ayy, ayz], [axz, ayz, azz]]
    one = jnp.ones_like(axx)
    zer = jnp.zeros_like(axx)
    Vm = [[one, zer, zer], [zer, one, zer], [zer, zer, one]]
    for _ in range(6):
        _jacobi_rotate(Am, Vm, 0, 1)
        _jacobi_rotate(Am, Vm, 0, 2)
        _jacobi_rotate(Am, Vm, 1, 2)

    wa, wb, wc = Am[0][0], Am[1][1], Am[2][2]
    w2 = jnp.maximum(jnp.maximum(wa, wb), wc)
    w0 = jnp.minimum(jnp.minimum(wa, wb), wc)
    w1 = wa + wb + wc - w2 - w0

    a_max = jnp.logical_and(wa >= wb, wa >= wc)
    b_max = jnp.logical_and(jnp.logical_not(a_max), wb >= wc)
    v2x = jnp.where(a_max, Vm[0][0], jnp.where(b_max, Vm[0][1], Vm[0][2]))
    v2y = jnp.where(a_max, Vm[1][0], jnp.where(b_max, Vm[1][1], Vm[1][2]))
    v2z = jnp.where(a_max, Vm[2][0], jnp.where(b_max, Vm[2][1], Vm[2][2]))

    dirwt = 1.0 - w1 / w2
    iw2 = 1.0 / w2

    x0 = Xc * v2x + Yc * v2y + Zc * v2z
    r2 = Xc * Xc + Yc * Yc + Zc * Zc - x0 * x0
    np0 = jnp.sqrt(jnp.maximum(r2, 0.0))
    sc = jnp.sum(x0 * np0, axis=1, keepdims=True)
    flip = jnp.where(sc < 0.0, -dirwt, dirwt)
    v0x = flip * v2x
    v0y = flip * v2y
    v0z = flip * v2z

    size = jnp.full_like(axx, float(S))
    o_ref[...] = jnp.concatenate(
        [cx, cy, cz,
         axx * iw2, axy * iw2, axz * iw2,
         axy * iw2, ayy * iw2, ayz * iw2,
         axz * iw2, ayz * iw2, azz * iw2,
         v0x, v0y, v0z, size],
        axis=1,
    )


def _tc_feats(xg, yg, zg):
    n = xg.shape[0]
    return pl.pallas_call(
        _feats_body,
        out_shape=jax.ShapeDtypeStruct((n, 16), jnp.float32),
    )(xg, yg, zg)


def kernel(data, clusts):
    n_clusts, S = clusts.shape
    voxels = data[:, 0:3].astype(jnp.float32)
    coords_t = voxels.T
    idx = clusts.reshape(-1).astype(jnp.int32)
    gx, gy, gz = _sc_gather(coords_t[0], coords_t[1], coords_t[2], idx)
    xg = gx.reshape(n_clusts, S)
    yg = gy.reshape(n_clusts, S)
    zg = gz.reshape(n_clusts, S)
    return _tc_feats(xg, yg, zg)


# speedup vs baseline: 1.9724x; 1.0543x over previous
import functools

import jax
import jax.numpy as jnp
from jax import lax
from jax.experimental import pallas as pl
from jax.experimental.pallas import tpu as pltpu
from jax.experimental.pallas import tpu_sc as plsc

_NC = 2
_NS = 16
_NW = _NC * _NS
_CHUNK = 128


def _sc_gather(t3, idx):
    B = idx.shape[0]
    N = t3.shape[2]
    b_per_w = B // _NW
    n_chunks = b_per_w // _CHUNK
    n_stage = N // _NS

    mesh = plsc.VectorSubcoreMesh(core_axis_name="c", subcore_axis_name="s")

    @functools.partial(
        pl.kernel,
        mesh=mesh,
        out_type=[jax.ShapeDtypeStruct((B,), jnp.float32)] * 3,
        scratch_types=[
            pltpu.VMEM_SHARED((N,), jnp.float32),
            pltpu.VMEM_SHARED((N,), jnp.float32),
            pltpu.VMEM_SHARED((N,), jnp.float32),
            pltpu.VMEM((b_per_w,), jnp.int32),
            pltpu.VMEM((b_per_w,), jnp.float32),
            pltpu.VMEM((b_per_w,), jnp.float32),
            pltpu.VMEM((b_per_w,), jnp.float32),
            pltpu.SemaphoreType.DMA,
            pltpu.SemaphoreType.DMA,
        ],
    )
    def gather_kernel(t3_hbm, idx_hbm, ox, oy, oz,
                      spx, spy, spz, idx_v, bx, by, bz, sem, osem):
        cid = lax.axis_index("c")
        sid = lax.axis_index("s")
        wid = sid * _NC + cid
        base = wid * b_per_w
        st = sid * n_stage
        stage = [
            pltpu.async_copy(t3_hbm.at[0, 0, pl.ds(st, n_stage)],
                             spx.at[pl.ds(st, n_stage)], sem),
            pltpu.async_copy(t3_hbm.at[1, 0, pl.ds(st, n_stage)],
                             spy.at[pl.ds(st, n_stage)], sem),
            pltpu.async_copy(t3_hbm.at[2, 0, pl.ds(st, n_stage)],
                             spz.at[pl.ds(st, n_stage)], sem),
        ]
        pltpu.sync_copy(idx_hbm.at[pl.ds(base, b_per_w)], idx_v)
        for cp in stage:
            cp.wait()
        plsc.subcore_barrier()
        copies = []
        for tab, buf in ((spx, bx), (spy, by), (spz, bz)):
            for j in range(n_chunks):
                sl = pl.ds(j * _CHUNK, _CHUNK)
                copies.append(pltpu.async_copy(tab.at[idx_v.at[sl]],
                                               buf.at[sl], sem))
        for cp in copies:
            cp.wait()
        outs = [
            pltpu.async_copy(bx, ox.at[pl.ds(base, b_per_w)], osem),
            pltpu.async_copy(by, oy.at[pl.ds(base, b_per_w)], osem),
            pltpu.async_copy(bz, oz.at[pl.ds(base, b_per_w)], osem),
        ]
        for cp in outs:
            cp.wait()

    return gather_kernel(t3, idx)


def _jacobi_rotate(Am, Vm, p, q):
    r = 3 - p - q
    app, aqq, apq = Am[p][p], Am[q][q], Am[p][q]
    apr, aqr = Am[p][r], Am[q][r]

    apq_zero = apq == 0.0
    apq_safe = jnp.where(apq_zero, 1.0, apq)
    tau = (aqq - app) * 0.5 / apq_safe
    sgn = jnp.where(tau >= 0.0, 1.0, -1.0)
    t = sgn / (jnp.abs(tau) + jnp.sqrt(1.0 + tau * tau))
    t = jnp.where(apq_zero, 0.0, t)
    c = lax.rsqrt(1.0 + t * t)
    s = t * c

    Am[p][p] = app - t * apq
    Am[q][q] = aqq + t * apq
    zero = apq * 0.0
    Am[p][q] = zero
    Am[q][p] = zero
    npr = c * apr - s * aqr
    nqr = c * aqr + s * apr
    Am[p][r] = npr
    Am[r][p] = npr
    Am[q][r] = nqr
    Am[r][q] = nqr
    for i in range(3):
        vip, viq = Vm[i][p], Vm[i][q]
        Vm[i][p] = c * vip - s * viq
        Vm[i][q] = s * vip + c * viq


def _feats_body(n_clusts, S, x_ref, y_ref, z_ref, o_ref):
    R = S // 128
    X = x_ref[...].reshape(n_clusts, R, 128)
    Y = y_ref[...].reshape(n_clusts, R, 128)
    Z = z_ref[...].reshape(n_clusts, R, 128)
    inv = jnp.float32(1.0 / S)

    def csum(v):
        return jnp.sum(v, axis=(1, 2), keepdims=True)

    cx = csum(X) * inv
    cy = csum(Y) * inv
    cz = csum(Z) * inv
    Xc = X - cx
    Yc = Y - cy
    Zc = Z - cz

    axx = csum(Xc * Xc)
    ayy = csum(Yc * Yc)
    azz = csum(Zc * Zc)
    axy = csum(Xc * Yc)
    axz = csum(Xc * Zc)
    ayz = csum(Yc * Zc)

    Am = [[axx, axy, axz], [axy, ayy, ayz], [axz, ayz, azz]]
    one = jnp.ones_like(axx)
    zer = jnp.zeros_like(axx)
    Vm = [[one, zer, zer], [zer, one, zer], [zer, zer, one]]
    for _ in range(6):
        _jacobi_rotate(Am, Vm, 0, 1)
        _jacobi_rotate(Am, Vm, 0, 2)
        _jacobi_rotate(Am, Vm, 1, 2)

    wa, wb, wc = Am[0][0], Am[1][1], Am[2][2]
    w2 = jnp.maximum(jnp.maximum(wa, wb), wc)
    w0 = jnp.minimum(jnp.minimum(wa, wb), wc)
    w1 = wa + wb + wc - w2 - w0

    a_max = jnp.logical_and(wa >= wb, wa >= wc)
    b_max = jnp.logical_and(jnp.logical_not(a_max), wb >= wc)
    v2x = jnp.where(a_max, Vm[0][0], jnp.where(b_max, Vm[0][1], Vm[0][2]))
    v2y = jnp.where(a_max, Vm[1][0], jnp.where(b_max, Vm[1][1], Vm[1][2]))
    v2z = jnp.where(a_max, Vm[2][0], jnp.where(b_max, Vm[2][1], Vm[2][2]))

    dirwt = 1.0 - w1 / w2
    iw2 = 1.0 / w2

    x0 = Xc * v2x + Yc * v2y + Zc * v2z
    r2 = Xc * Xc + Yc * Yc + Zc * Zc - x0 * x0
    np0 = jnp.sqrt(jnp.maximum(r2, 0.0))
    sc = csum(x0 * np0)
    flip = jnp.where(sc < 0.0, -dirwt, dirwt)
    v0x = flip * v2x
    v0y = flip * v2y
    v0z = flip * v2z

    size = jnp.full_like(axx, float(S))
    row = jnp.concatenate(
        [cx, cy, cz,
         axx * iw2, axy * iw2, axz * iw2,
         axy * iw2, ayy * iw2, ayz * iw2,
         axz * iw2, ayz * iw2, azz * iw2,
         v0x, v0y, v0z, size],
        axis=2,
    )
    o_ref[...] = row.reshape(n_clusts, 16)


def _tc_feats(gx, gy, gz, n_clusts, S):
    body = functools.partial(_feats_body, n_clusts, S)
    return pl.pallas_call(
        body,
        out_shape=jax.ShapeDtypeStruct((n_clusts, 16), jnp.float32),
    )(gx, gy, gz)


def kernel(data, clusts):
    n_clusts, S = clusts.shape
    voxels = data[:, 0:3].astype(jnp.float32)
    t3 = voxels.T.reshape(3, 1, -1)
    idx = clusts.reshape(-1).astype(jnp.int32)
    gx, gy, gz = _sc_gather(t3, idx)
    rows = (n_clusts * S) // 128
    return _tc_feats(gx.reshape(rows, 128), gy.reshape(rows, 128),
                     gz.reshape(rows, 128), n_clusts, S)


# speedup vs baseline: 1.9881x; 1.0080x over previous
import functools

import jax
import jax.numpy as jnp
from jax import lax
from jax.experimental import pallas as pl
from jax.experimental.pallas import tpu as pltpu
from jax.experimental.pallas import tpu_sc as plsc

_NC = 2
_NS = 16
_NW = _NC * _NS
_CHUNK = 128


def _sc_gather(t3, idx):
    B = idx.shape[0]
    N = t3.shape[2]
    b_per_w = B // _NW
    n_chunks = b_per_w // _CHUNK
    n_stage = N // _NS

    mesh = plsc.VectorSubcoreMesh(core_axis_name="c", subcore_axis_name="s")

    @functools.partial(
        pl.kernel,
        mesh=mesh,
        out_type=[jax.ShapeDtypeStruct((B,), jnp.float32)] * 3,
        scratch_types=[
            pltpu.VMEM_SHARED((N,), jnp.float32),
            pltpu.VMEM_SHARED((N,), jnp.float32),
            pltpu.VMEM_SHARED((N,), jnp.float32),
            pltpu.VMEM((b_per_w,), jnp.int32),
            pltpu.VMEM((b_per_w,), jnp.float32),
            pltpu.VMEM((b_per_w,), jnp.float32),
            pltpu.VMEM((b_per_w,), jnp.float32),
            pltpu.SemaphoreType.DMA,
            pltpu.SemaphoreType.DMA,
        ],
    )
    def gather_kernel(t3_hbm, idx_hbm, ox, oy, oz,
                      spx, spy, spz, idx_v, bx, by, bz, sem, osem):
        cid = lax.axis_index("c")
        sid = lax.axis_index("s")
        wid = sid * _NC + cid
        base = wid * b_per_w
        st = sid * n_stage
        stage = [
            pltpu.async_copy(t3_hbm.at[0, 0, pl.ds(st, n_stage)],
                             spx.at[pl.ds(st, n_stage)], sem),
            pltpu.async_copy(t3_hbm.at[1, 0, pl.ds(st, n_stage)],
                             spy.at[pl.ds(st, n_stage)], sem),
            pltpu.async_copy(t3_hbm.at[2, 0, pl.ds(st, n_stage)],
                             spz.at[pl.ds(st, n_stage)], sem),
        ]
        pltpu.sync_copy(idx_hbm.at[pl.ds(base, b_per_w)], idx_v)
        for cp in stage:
            cp.wait()
        plsc.subcore_barrier()
        copies = []
        for tab, buf in ((spx, bx), (spy, by), (spz, bz)):
            per = []
            for j in range(n_chunks):
                sl = pl.ds(j * _CHUNK, _CHUNK)
                per.append(pltpu.async_copy(tab.at[idx_v.at[sl]],
                                            buf.at[sl], sem))
            copies.append(per)
        outs = []
        for per, buf, dst in ((copies[0], bx, ox), (copies[1], by, oy),
                              (copies[2], bz, oz)):
            for cp in per:
                cp.wait()
            outs.append(pltpu.async_copy(buf, dst.at[pl.ds(base, b_per_w)],
                                         osem))
        for cp in outs:
            cp.wait()

    return gather_kernel(t3, idx)


def _jacobi_rotate(Am, Vm, p, q):
    r = 3 - p - q
    app, aqq, apq = Am[p][p], Am[q][q], Am[p][q]
    apr, aqr = Am[p][r], Am[q][r]

    apq_zero = apq == 0.0
    apq_safe = jnp.where(apq_zero, 1.0, apq)
    tau = (aqq - app) * 0.5 / apq_safe
    sgn = jnp.where(tau >= 0.0, 1.0, -1.0)
    t = sgn / (jnp.abs(tau) + jnp.sqrt(1.0 + tau * tau))
    t = jnp.where(apq_zero, 0.0, t)
    c = lax.rsqrt(1.0 + t * t)
    s = t * c

    Am[p][p] = app - t * apq
    Am[q][q] = aqq + t * apq
    zero = apq * 0.0
    Am[p][q] = zero
    Am[q][p] = zero
    npr = c * apr - s * aqr
    nqr = c * aqr + s * apr
    Am[p][r] = npr
    Am[r][p] = npr
    Am[q][r] = nqr
    Am[r][q] = nqr
    for i in range(3):
        vip, viq = Vm[i][p], Vm[i][q]
        Vm[i][p] = c * vip - s * viq
        Vm[i][q] = s * vip + c * viq


def _feats_body(n_clusts, S, x_ref, y_ref, z_ref, o_ref):
    R = S // 128
    X = x_ref[...]
    Y = y_ref[...]
    Z = z_ref[...]
    inv = jnp.float32(1.0 / S)

    def csum(v):
        r = jnp.sum(v, axis=1, keepdims=True).reshape(n_clusts, R, 1)
        return jnp.sum(r, axis=1)

    def rep(q):
        return jnp.broadcast_to(q[:, None, :],
                                (n_clusts, R, 1)).reshape(n_clusts * R, 1)

    cx = csum(X) * inv
    cy = csum(Y) * inv
    cz = csum(Z) * inv
    Xc = X - rep(cx)
    Yc = Y - rep(cy)
    Zc = Z - rep(cz)

    axx = csum(Xc * Xc)
    ayy = csum(Yc * Yc)
    azz = csum(Zc * Zc)
    axy = csum(Xc * Yc)
    axz = csum(Xc * Zc)
    ayz = csum(Yc * Zc)

    Am = [[axx, axy, axz], [axy, ayy, ayz], [axz, ayz, azz]]
    one = jnp.ones_like(axx)
    zer = jnp.zeros_like(axx)
    Vm = [[one, zer, zer], [zer, one, zer], [zer, zer, one]]
    for _ in range(6):
        _jacobi_rotate(Am, Vm, 0, 1)
        _jacobi_rotate(Am, Vm, 0, 2)
        _jacobi_rotate(Am, Vm, 1, 2)

    wa, wb, wc = Am[0][0], Am[1][1], Am[2][2]
    w2 = jnp.maximum(jnp.maximum(wa, wb), wc)
    w0 = jnp.minimum(jnp.minimum(wa, wb), wc)
    w1 = wa + wb + wc - w2 - w0

    a_max = jnp.logical_and(wa >= wb, wa >= wc)
    b_max = jnp.logical_and(jnp.logical_not(a_max), wb >= wc)
    v2x = jnp.where(a_max, Vm[0][0], jnp.where(b_max, Vm[0][1], Vm[0][2]))
    v2y = jnp.where(a_max, Vm[1][0], jnp.where(b_max, Vm[1][1], Vm[1][2]))
    v2z = jnp.where(a_max, Vm[2][0], jnp.where(b_max, Vm[2][1], Vm[2][2]))

    dirwt = 1.0 - w1 / w2
    iw2 = 1.0 / w2

    x0 = Xc * rep(v2x) + Yc * rep(v2y) + Zc * rep(v2z)
    r2 = Xc * Xc + Yc * Yc + Zc * Zc - x0 * x0
    np0 = jnp.sqrt(jnp.maximum(r2, 0.0))
    sc = csum(x0 * np0)
    flip = jnp.where(sc < 0.0, -dirwt, dirwt)
    v0x = flip * v2x
    v0y = flip * v2y
    v0z = flip * v2z

    size = jnp.full_like(axx, float(S))
    o_ref[...] = jnp.concatenate(
        [cx, cy, cz,
         axx * iw2, axy * iw2, axz * iw2,
         axy * iw2, ayy * iw2, ayz * iw2,
         axz * iw2, ayz * iw2, azz * iw2,
         v0x, v0y, v0z, size],
        axis=1,
    )


def _tc_feats(gx, gy, gz, n_clusts, S):
    body = functools.partial(_feats_body, n_clusts, S)
    return pl.pallas_call(
        body,
        out_shape=jax.ShapeDtypeStruct((n_clusts, 16), jnp.float32),
    )(gx, gy, gz)


def kernel(data, clusts):
    n_clusts, S = clusts.shape
    voxels = data[:, 0:3].astype(jnp.float32)
    t3 = voxels.T.reshape(3, 1, -1)
    idx = clusts.reshape(-1).astype(jnp.int32)
    gx, gy, gz = _sc_gather(t3, idx)
    rows = (n_clusts * S) // 128
    return _tc_feats(gx.reshape(rows, 128), gy.reshape(rows, 128),
                     gz.reshape(rows, 128), n_clusts, S)
